# baseline (speedup 1.0000x reference)
import jax
import jax.numpy as jnp
from jax.experimental import pallas as pl
from jax.experimental.pallas import tpu as pltpu

_B, _T = 128, 8192
_K = 5
_W = 512
_NB = _T // _W
_CW = _NB * 8
_NEG = float("-inf")


def _fast_body(snip_ref, anom_ref, lab_ref, loss_ref, flag_ref,
               mval_ref, mcnt_ref, msum_ref, dacc_ref):
    j = pl.program_id(0)
    p = snip_ref[...]
    a = anom_ref[...]

    log1mp = jnp.maximum(jnp.log1p(-p), -100.0)
    part = jnp.sum(log1mp)

    @pl.when(j == 0)
    def _init():
        dacc_ref[0, 0] = 0.0

    dacc_ref[0, 0] = dacc_ref[0, 0] + part

    pad = jnp.full((_B, 8 - _K), _NEG, jnp.float32)
    zpad = jnp.zeros((_B, 8 - _K), jnp.float32)
    ms, cnts, ssums = [], [], []
    for k in range(_K):
        m = jnp.max(a, axis=1, keepdims=True)
        eq = a == m
        cnt = jnp.sum(jnp.where(eq, 1.0, 0.0), axis=1, keepdims=True)
        ssum = jnp.sum(jnp.where(eq, p, 0.0), axis=1, keepdims=True)
        a = jnp.where(eq, _NEG, a)
        ms.append(m)
        cnts.append(cnt)
        ssums.append(ssum)
    mval_ref[j] = jnp.concatenate(ms + [pad], axis=1)
    mcnt_ref[j] = jnp.concatenate(cnts + [zpad], axis=1)
    msum_ref[j] = jnp.concatenate(ssums + [zpad], axis=1)

    @pl.when(j == _NB - 1)
    def _merge():
        cv = jnp.concatenate([mval_ref[jj] for jj in range(_NB)], axis=1)
        cc = jnp.concatenate([mcnt_ref[jj] for jj in range(_NB)], axis=1)
        cs = jnp.concatenate([msum_ref[jj] for jj in range(_NB)], axis=1)
        fake = lab_ref[...] == 1
        corr = jnp.float32(0.0)
        bad = jnp.float32(0.0)
        v = cv
        for _ in range(_K):
            gm = jnp.max(v, axis=1, keepdims=True)
            geq = v == gm
            gc = jnp.sum(jnp.where(geq, cc, 0.0), axis=1, keepdims=True)
            gs = jnp.sum(jnp.where(geq, cs, 0.0), axis=1, keepdims=True)
            v = jnp.where(geq, _NEG, v)
            l1 = jnp.maximum(jnp.log1p(-gs), -100.0)
            lp = jnp.maximum(jnp.log(gs), -100.0)
            corr = corr + jnp.sum(jnp.where(fake, l1 - lp, 0.0))
            bad = bad + jnp.sum(jnp.where(gc != 1.0, 1.0, 0.0))
        dense = dacc_ref[0, 0]
        loss = (corr - dense) / (_B * _T)
        loss_ref[...] = jnp.broadcast_to(loss, (1, 1))
        flag_ref[...] = jnp.broadcast_to((bad > 0.0).astype(jnp.int32), (1, 1))


def _exact_body(snip_ref, anom_ref, lab_ref, out_ref):
    p = snip_ref[...]
    a = anom_ref[...]
    fake = lab_ref[...] == 1

    log1mp = jnp.maximum(jnp.log1p(-p), -100.0)
    dense = jnp.sum(log1mp)

    col = jax.lax.broadcasted_iota(jnp.int32, (_B, _T), 1)
    corr = jnp.float32(0.0)
    for _ in range(_K):
        m = jnp.max(a, axis=1, keepdims=True)
        eq = a == m
        idx = jnp.min(jnp.where(eq, col, _T), axis=1, keepdims=True)
        hit = col == idx
        s = jnp.sum(jnp.where(hit, p, 0.0), axis=1, keepdims=True)
        l1 = jnp.maximum(jnp.log1p(-s), -100.0)
        lp = jnp.maximum(jnp.log(s), -100.0)
        corr = corr + jnp.sum(jnp.where(fake, l1 - lp, 0.0))
        a = jnp.where(hit, _NEG, a)

    loss = (corr - dense) / (_B * _T)
    out_ref[...] = jnp.broadcast_to(loss, (1, 1))


def kernel(snippet_scores, anomaly_scores, labels):
    lab2d = labels.reshape(_B, 1)

    loss_fast, flag = pl.pallas_call(
        _fast_body,
        grid=(_NB,),
        in_specs=[
            pl.BlockSpec((_B, _W), lambda j: (0, j)),
            pl.BlockSpec((_B, _W), lambda j: (0, j)),
            pl.BlockSpec((_B, 1), lambda j: (0, 0)),
        ],
        out_specs=[
            pl.BlockSpec((1, 1), lambda j: (0, 0)),
            pl.BlockSpec((1, 1), lambda j: (0, 0)),
        ],
        out_shape=[
            jax.ShapeDtypeStruct((1, 1), jnp.float32),
            jax.ShapeDtypeStruct((1, 1), jnp.int32),
        ],
        scratch_shapes=[
            pltpu.VMEM((_NB, _B, 8), jnp.float32),
            pltpu.VMEM((_NB, _B, 8), jnp.float32),
            pltpu.VMEM((_NB, _B, 8), jnp.float32),
            pltpu.SMEM((1, 1), jnp.float32),
        ],
    )(snippet_scores, anomaly_scores, lab2d)

    def _slow():
        return pl.pallas_call(
            _exact_body,
            out_shape=jax.ShapeDtypeStruct((1, 1), jnp.float32),
        )(snippet_scores, anomaly_scores, lab2d)

    loss = jax.lax.cond(flag[0, 0] != 0, _slow, lambda: loss_fast)
    return loss.reshape(1)


# speedup vs baseline: 1.4222x; 1.4222x over previous
import jax
import jax.numpy as jnp
from jax.experimental import pallas as pl
from jax.experimental.pallas import tpu as pltpu

_B, _T = 128, 8192
_K = 5
_W = 2048
_NB = _T // _W
_CW = _NB * 8
_NEG = float("-inf")


def _fast_body(snip_ref, anom_ref, lab_ref, loss_ref, flag_ref,
               mval_ref, mcnt_ref, msum_ref, dacc_ref):
    j = pl.program_id(0)
    p = snip_ref[...]
    a = anom_ref[...]

    log1mp = jnp.maximum(jnp.log1p(-p), -100.0)
    part = jnp.sum(log1mp)

    @pl.when(j == 0)
    def _init():
        dacc_ref[0, 0] = 0.0

    dacc_ref[0, 0] = dacc_ref[0, 0] + part

    pad = jnp.full((_B, 8 - _K), _NEG, jnp.float32)
    zpad = jnp.zeros((_B, 8 - _K), jnp.float32)
    ms, cnts, ssums = [], [], []
    for k in range(_K):
        m = jnp.max(a, axis=1, keepdims=True)
        eq = a == m
        cnt = jnp.sum(jnp.where(eq, 1.0, 0.0), axis=1, keepdims=True)
        ssum = jnp.sum(jnp.where(eq, p, 0.0), axis=1, keepdims=True)
        a = jnp.where(eq, _NEG, a)
        ms.append(m)
        cnts.append(cnt)
        ssums.append(ssum)
    mval_ref[j] = jnp.concatenate(ms + [pad], axis=1)
    mcnt_ref[j] = jnp.concatenate(cnts + [zpad], axis=1)
    msum_ref[j] = jnp.concatenate(ssums + [zpad], axis=1)

    @pl.when(j == _NB - 1)
    def _merge():
        cv = jnp.concatenate([mval_ref[jj] for jj in range(_NB)], axis=1)
        cc = jnp.concatenate([mcnt_ref[jj] for jj in range(_NB)], axis=1)
        cs = jnp.concatenate([msum_ref[jj] for jj in range(_NB)], axis=1)
        fake = lab_ref[...] == 1
        corr = jnp.float32(0.0)
        bad = jnp.float32(0.0)
        v = cv
        for _ in range(_K):
            gm = jnp.max(v, axis=1, keepdims=True)
            geq = v == gm
            gc = jnp.sum(jnp.where(geq, cc, 0.0), axis=1, keepdims=True)
            gs = jnp.sum(jnp.where(geq, cs, 0.0), axis=1, keepdims=True)
            v = jnp.where(geq, _NEG, v)
            l1 = jnp.maximum(jnp.log1p(-gs), -100.0)
            lp = jnp.maximum(jnp.log(gs), -100.0)
            corr = corr + jnp.sum(jnp.where(fake, l1 - lp, 0.0))
            bad = bad + jnp.sum(jnp.where(gc != 1.0, 1.0, 0.0))
        dense = dacc_ref[0, 0]
        loss = (corr - dense) / (_B * _T)
        loss_ref[...] = jnp.broadcast_to(loss, (1, 1))
        flag_ref[...] = jnp.broadcast_to((bad > 0.0).astype(jnp.int32), (1, 1))


def _exact_body(snip_ref, anom_ref, lab_ref, out_ref):
    p = snip_ref[...]
    a = anom_ref[...]
    fake = lab_ref[...] == 1

    log1mp = jnp.maximum(jnp.log1p(-p), -100.0)
    dense = jnp.sum(log1mp)

    col = jax.lax.broadcasted_iota(jnp.int32, (_B, _T), 1)
    corr = jnp.float32(0.0)
    for _ in range(_K):
        m = jnp.max(a, axis=1, keepdims=True)
        eq = a == m
        idx = jnp.min(jnp.where(eq, col, _T), axis=1, keepdims=True)
        hit = col == idx
        s = jnp.sum(jnp.where(hit, p, 0.0), axis=1, keepdims=True)
        l1 = jnp.maximum(jnp.log1p(-s), -100.0)
        lp = jnp.maximum(jnp.log(s), -100.0)
        corr = corr + jnp.sum(jnp.where(fake, l1 - lp, 0.0))
        a = jnp.where(hit, _NEG, a)

    loss = (corr - dense) / (_B * _T)
    out_ref[...] = jnp.broadcast_to(loss, (1, 1))


def kernel(snippet_scores, anomaly_scores, labels):
    lab2d = labels.reshape(_B, 1)

    loss_fast, flag = pl.pallas_call(
        _fast_body,
        grid=(_NB,),
        in_specs=[
            pl.BlockSpec((_B, _W), lambda j: (0, j)),
            pl.BlockSpec((_B, _W), lambda j: (0, j)),
            pl.BlockSpec((_B, 1), lambda j: (0, 0)),
        ],
        out_specs=[
            pl.BlockSpec((1, 1), lambda j: (0, 0)),
            pl.BlockSpec((1, 1), lambda j: (0, 0)),
        ],
        out_shape=[
            jax.ShapeDtypeStruct((1, 1), jnp.float32),
            jax.ShapeDtypeStruct((1, 1), jnp.int32),
        ],
        scratch_shapes=[
            pltpu.VMEM((_NB, _B, 8), jnp.float32),
            pltpu.VMEM((_NB, _B, 8), jnp.float32),
            pltpu.VMEM((_NB, _B, 8), jnp.float32),
            pltpu.SMEM((1, 1), jnp.float32),
        ],
    )(snippet_scores, anomaly_scores, lab2d)

    def _slow():
        return pl.pallas_call(
            _exact_body,
            out_shape=jax.ShapeDtypeStruct((1, 1), jnp.float32),
        )(snippet_scores, anomaly_scores, lab2d)

    loss = jax.lax.cond(flag[0, 0] != 0, _slow, lambda: loss_fast)
    return loss.reshape(1)
